# baseline (speedup 1.0000x reference)
import functools

import jax
import jax.numpy as jnp
from jax import lax
from jax.experimental import pallas as pl
from jax.experimental.pallas import tpu as pltpu
from jax.experimental.pallas import tpu_sc as plsc

_NUM_USERS = 100000
_D = 32
_B = 16384
_NC = 2
_NS = 16
_L = 16
_NW = _NC * _NS
_BPW = _B // _NW
_KB = _BPW // 128

_mesh = plsc.VectorSubcoreMesh(core_axis_name="c", subcore_axis_name="s")


@functools.partial(
    pl.kernel,
    mesh=_mesh,
    compiler_params=pltpu.CompilerParams(
        needs_layout_passes=False, use_tc_tiling_on_sc=False),
    out_type=jax.ShapeDtypeStruct((_B,), jnp.float32),
    scratch_types=[
        pltpu.VMEM((_KB, 128), jnp.int32),
        pltpu.VMEM((_KB, 128), jnp.int32),
        pltpu.VMEM((_BPW, _D), jnp.float32),
        pltpu.VMEM((_BPW, _D), jnp.float32),
        pltpu.VMEM((_D + _L,), jnp.float32),
        pltpu.VMEM((_BPW,), jnp.float32),
        pltpu.SemaphoreType.DMA,
        pltpu.SemaphoreType.DMA,
    ],
)
def _mf_sc(users_hbm, items_hbm, utab_hbm, itab_hbm, params_hbm,
           out_hbm, uidx, iidx, urows, irows, params, outv, sem_u, sem_i):
    wid = lax.axis_index("s") * _NC + lax.axis_index("c")

    pltpu.sync_copy(users_hbm.at[pl.ds(wid * _KB, _KB)], uidx)
    ucopies = [
        pltpu.async_copy(utab_hbm.at[uidx.at[k]],
                         urows.at[pl.ds(k * 128, 128)], sem_u)
        for k in range(_KB)
    ]

    pltpu.sync_copy(items_hbm.at[pl.ds(wid * _KB, _KB)], iidx)
    for k in range(_KB):
        for o in range(128 // _L):
            iidx[k, pl.ds(o * _L, _L)] = iidx[k, pl.ds(o * _L, _L)] - _NUM_USERS
    icopies = [
        pltpu.async_copy(itab_hbm.at[iidx.at[k]],
                         irows.at[pl.ds(k * 128, 128)], sem_i)
        for k in range(_KB)
    ]

    pltpu.sync_copy(params_hbm, params)
    for cp in ucopies:
        cp.wait()
    for cp in icopies:
        cp.wait()

    lane = jnp.arange(_L, dtype=jnp.int32)
    w0 = params[pl.ds(0, _L)]
    w1 = params[pl.ds(_L, _L)]
    bias = params[pl.ds(_D, _L)]

    def chunk(t, carry):
        r = bias
        for jj in range(_L):
            j = t * _L + jj
            p = (urows[j, pl.ds(0, _L)] * irows[j, pl.ds(0, _L)] * w0
                 + urows[j, pl.ds(_L, _L)] * irows[j, pl.ds(_L, _L)] * w1)
            r = jnp.where(lane == jj, r + jnp.sum(p), r)
        outv[pl.ds(t * _L, _L)] = 1.0 / (1.0 + jnp.exp(-r))
        return carry

    lax.fori_loop(0, _BPW // _L, chunk, 0)
    pltpu.sync_copy(outv, out_hbm.at[pl.ds(wid * _BPW, _BPW)])


def kernel(users, items, user_table, item_table, W, b):
    users2d = users.reshape(_B // 128, 128)
    items2d = items.reshape(_B // 128, 128)
    params = jnp.concatenate(
        [W.reshape(-1), jnp.full((_L,), b[0], dtype=jnp.float32)])
    return _mf_sc(users2d, items2d, user_table, item_table, params)


# speedup vs baseline: 2.8264x; 2.8264x over previous
import functools

import jax
import jax.numpy as jnp
from jax import lax
from jax.experimental import pallas as pl
from jax.experimental.pallas import tpu as pltpu
from jax.experimental.pallas import tpu_sc as plsc

_NUM_USERS = 100000
_NUM_ITEMS = 1000000
_D = 32
_B = 16384
_NC = 2
_NS = 16
_L = 16
_NW = _NC * _NS
_BPW = _B // _NW
_KB = _BPW // 128
_NPAN = _NUM_ITEMS // 128
_TAIL0 = _NPAN * 128
_G = 16
_NG = _BPW // _G

_mesh = plsc.VectorSubcoreMesh(core_axis_name="c", subcore_axis_name="s")


@functools.partial(
    pl.kernel,
    mesh=_mesh,
    compiler_params=pltpu.CompilerParams(needs_layout_passes=False),
    out_type=jax.ShapeDtypeStruct((_B,), jnp.float32),
    scratch_types=[
        pltpu.VMEM((_KB, 128), jnp.int32),
        pltpu.VMEM((_KB, 128), jnp.int32),
        pltpu.VMEM((_G, 128), jnp.float32),
        pltpu.VMEM((_G, 4, 8, 128), jnp.float32),
        pltpu.VMEM((64, _D), jnp.float32),
        pltpu.VMEM((_D + _L,), jnp.float32),
        pltpu.VMEM((_BPW,), jnp.float32),
        pltpu.SemaphoreType.DMA,
        pltpu.SemaphoreType.DMA,
    ],
)
def _mf_sc(users_hbm, items_hbm, utab4_hbm, it3_hbm, itail_hbm, params_hbm,
           out_hbm, uidx, iidx, ugrp, ipan, itail, params, outv,
           sem_u, sem_i):
    wid = lax.axis_index("s") * _NC + lax.axis_index("c")

    pltpu.sync_copy(users_hbm.at[pl.ds(wid * _KB, _KB)], uidx)
    pltpu.sync_copy(items_hbm.at[pl.ds(wid * _KB, _KB)], iidx)
    for k in range(_KB):
        for o in range(128 // _L):
            iidx[k, pl.ds(o * _L, _L)] = iidx[k, pl.ds(o * _L, _L)] - _NUM_USERS
    pltpu.sync_copy(itail_hbm, itail)
    pltpu.sync_copy(params_hbm, params)

    lane = jnp.arange(_L, dtype=jnp.int32)
    w0 = params[pl.ds(0, _L)]
    w1 = params[pl.ds(_L, _L)]
    bias = params[pl.ds(_D, _L)]
    tv0 = lane >> 3
    rv = lane & 7
    tv1 = tv0 + 2

    def body(g, carry):
        uvec = uidx[g >> 3, pl.ds((g & 7) * _L, _L)]
        idvec = iidx[g >> 3, pl.ds((g & 7) * _L, _L)]
        ucp = pltpu.async_copy(utab4_hbm.at[uvec >> 2], ugrp, sem_u)
        icps = []
        for jj in range(_G):
            pan = jnp.minimum(idvec[jj] >> 7, _NPAN - 1)
            icps.append(pltpu.async_copy(
                it3_hbm.at[:, :, pl.ds(pan * 128, 128)], ipan.at[jj], sem_i))
        ucp.wait()
        for cp in icps:
            cp.wait()

        r_acc = bias
        for jj in range(_G):
            idj = idvec[jj]
            lcol = jnp.full((_L,), idj & 127, dtype=jnp.int32)
            jv = jnp.full((_L,), jj, dtype=jnp.int32)
            i0 = plsc.load_gather(ipan, [jv, tv0, rv, lcol])
            i1 = plsc.load_gather(ipan, [jv, tv1, rv, lcol])
            trow = jnp.full((_L,), jnp.clip(idj - _TAIL0, 0, 63),
                            dtype=jnp.int32)
            t0 = plsc.load_gather(itail, [trow, lane])
            t1 = plsc.load_gather(itail, [trow, lane + _L])
            is_tail = idj >= _TAIL0
            i0 = jnp.where(is_tail, t0, i0)
            i1 = jnp.where(is_tail, t1, i1)
            usub = (uvec[jj] & 3) * _D
            p = (ugrp[jj, pl.ds(usub, _L)] * i0 * w0
                 + ugrp[jj, pl.ds(usub + _L, _L)] * i1 * w1)
            r_acc = jnp.where(lane == jj, r_acc + jnp.sum(p), r_acc)
        outv[pl.ds(g * _G, _L)] = 1.0 / (1.0 + jnp.exp(-r_acc))
        return carry

    lax.fori_loop(0, _NG, body, 0)
    pltpu.sync_copy(outv, out_hbm.at[pl.ds(wid * _BPW, _BPW)])


def kernel(users, items, user_table, item_table, W, b):
    users2d = users.reshape(_B // 128, 128)
    items2d = items.reshape(_B // 128, 128)
    utab4 = user_table.reshape(_NUM_USERS // 4, 128)
    it3 = item_table.T.reshape(4, 8, _NUM_ITEMS)
    itail = item_table[_TAIL0:]
    params = jnp.concatenate(
        [W.reshape(-1), jnp.full((_L,), b[0], dtype=jnp.float32)])
    return _mf_sc(users2d, items2d, utab4, it3, itail, params)


# speedup vs baseline: 2.8463x; 1.0071x over previous
import functools

import jax
import jax.numpy as jnp
from jax import lax
from jax.experimental import pallas as pl
from jax.experimental.pallas import tpu as pltpu
from jax.experimental.pallas import tpu_sc as plsc

_NUM_USERS = 100000
_NUM_ITEMS = 1000000
_D = 32
_B = 16384
_NC = 2
_NS = 16
_L = 16
_NW = _NC * _NS
_BPW = _B // _NW
_KB = _BPW // 128
_NPAN = _NUM_ITEMS // 128
_TAIL0 = _NPAN * 128
_G = 8
_NG = _BPW // _G

_mesh = plsc.VectorSubcoreMesh(core_axis_name="c", subcore_axis_name="s")


@functools.partial(
    pl.kernel,
    mesh=_mesh,
    compiler_params=pltpu.CompilerParams(needs_layout_passes=False),
    out_type=jax.ShapeDtypeStruct((_B,), jnp.float32),
    scratch_types=[
        pltpu.VMEM((_KB, 128), jnp.int32),
        pltpu.VMEM((_KB, 128), jnp.int32),
        pltpu.VMEM((_KB, 128), jnp.int32),
        pltpu.VMEM((2, _G, 128), jnp.float32),
        pltpu.VMEM((2, _G, 4, 8, 128), jnp.float32),
        pltpu.VMEM((64, _D), jnp.float32),
        pltpu.VMEM((_D + _L,), jnp.float32),
        pltpu.VMEM((_BPW,), jnp.float32),
        pltpu.SemaphoreType.DMA,
        pltpu.SemaphoreType.DMA,
        pltpu.SemaphoreType.DMA,
        pltpu.SemaphoreType.DMA,
    ],
)
def _mf_sc(users_hbm, items_hbm, utab4_hbm, it3_hbm, itail_hbm, params_hbm,
           out_hbm, uidx, iidx, uidx4, ugrp, ipan, itail, params, outv,
           sem_u0, sem_u1, sem_i0, sem_i1):
    wid = lax.axis_index("s") * _NC + lax.axis_index("c")
    sem_u = [sem_u0, sem_u1]
    sem_i = [sem_i0, sem_i1]

    pltpu.sync_copy(users_hbm.at[pl.ds(wid * _KB, _KB)], uidx)
    pltpu.sync_copy(items_hbm.at[pl.ds(wid * _KB, _KB)], iidx)
    for k in range(_KB):
        for o in range(128 // _L):
            iidx[k, pl.ds(o * _L, _L)] = iidx[k, pl.ds(o * _L, _L)] - _NUM_USERS
            uidx4[k, pl.ds(o * _L, _L)] = uidx[k, pl.ds(o * _L, _L)] >> 2
    pltpu.sync_copy(itail_hbm, itail)
    pltpu.sync_copy(params_hbm, params)

    lane = jnp.arange(_L, dtype=jnp.int32)
    w0 = params[pl.ds(0, _L)]
    w1 = params[pl.ds(_L, _L)]
    bias = params[pl.ds(_D, _L)]
    tv0 = lane >> 3
    rv = lane & 7
    tv1 = tv0 + 2

    def id_slices(g, par):
        blk = g >> 4
        off16 = ((g & 15) >> 1) * _L
        half = par * _G
        return (uidx[blk, pl.ds(off16, _L)],
                iidx[blk, pl.ds(off16, _L)], half)

    def fire(g, par):
        blk = g >> 4
        off8 = (g & 15) * _G
        pltpu.async_copy(
            utab4_hbm.at[uidx4.at[blk, pl.ds(off8, _G)]],
            ugrp.at[par], sem_u[par])
        _, ivec, half = id_slices(g, par)
        for jj in range(_G):
            pan = jnp.minimum(ivec[half + jj] >> 7, _NPAN - 1)
            pltpu.async_copy(
                it3_hbm.at[:, :, pl.ds(pan * 128, 128)],
                ipan.at[par, jj], sem_i[par])

    def drain(par):
        pltpu.make_async_copy(
            utab4_hbm.at[pl.ds(0, _G)], ugrp.at[par], sem_u[par]).wait()
        for jj in range(_G):
            pltpu.make_async_copy(
                it3_hbm.at[:, :, pl.ds(0, 128)],
                ipan.at[par, jj], sem_i[par]).wait()

    def process(g, par, r_acc):
        uvec, ivec, half = id_slices(g, par)
        for jj in range(_G):
            idj = ivec[half + jj]
            lcol = jnp.full((_L,), idj & 127, dtype=jnp.int32)
            jv = jnp.full((_L,), jj, dtype=jnp.int32)
            pv = jnp.full((_L,), par, dtype=jnp.int32)
            i0 = plsc.load_gather(ipan, [pv, jv, tv0, rv, lcol])
            i1 = plsc.load_gather(ipan, [pv, jv, tv1, rv, lcol])
            trow = jnp.full((_L,), jnp.clip(idj - _TAIL0, 0, 63),
                            dtype=jnp.int32)
            t0 = plsc.load_gather(itail, [trow, lane])
            t1 = plsc.load_gather(itail, [trow, lane + _L])
            is_tail = idj >= _TAIL0
            i0 = jnp.where(is_tail, t0, i0)
            i1 = jnp.where(is_tail, t1, i1)
            usub = (uvec[half + jj] & 3) * _D
            p = (ugrp[par, jj, pl.ds(usub, _L)] * i0 * w0
                 + ugrp[par, jj, pl.ds(usub + _L, _L)] * i1 * w1)
            r_acc = jnp.where(lane == half + jj, r_acc + jnp.sum(p), r_acc)
        return r_acc

    fire(0, 0)

    def body(gp, carry):
        g0 = gp * 2
        g1 = g0 + 1
        fire(g1, 1)
        drain(0)
        r_acc = process(g0, 0, bias)

        @pl.when(gp < _NG // 2 - 1)
        def _():
            fire(g0 + 2, 0)

        drain(1)
        r_acc = process(g1, 1, r_acc)
        outv[pl.ds(gp * _L, _L)] = 1.0 / (1.0 + jnp.exp(-r_acc))
        return carry

    lax.fori_loop(0, _NG // 2, body, 0)
    pltpu.sync_copy(outv, out_hbm.at[pl.ds(wid * _BPW, _BPW)])


def kernel(users, items, user_table, item_table, W, b):
    users2d = users.reshape(_B // 128, 128)
    items2d = items.reshape(_B // 128, 128)
    utab4 = user_table.reshape(_NUM_USERS // 4, 128)
    it3 = item_table.T.reshape(4, 8, _NUM_ITEMS)
    itail = item_table[_TAIL0:]
    params = jnp.concatenate(
        [W.reshape(-1), jnp.full((_L,), b[0], dtype=jnp.float32)])
    return _mf_sc(users2d, items2d, utab4, it3, itail, params)
